# baseline (speedup 1.0000x reference)
import functools

import jax
import jax.numpy as jnp
from jax import lax
from jax.experimental import pallas as pl
from jax.experimental.pallas import tpu as pltpu
from jax.experimental.pallas import tpu_sc as plsc

N = 10000
F = 128
E = 320000
NUM_HEADS = 4
HREP = 12
NPAD = 10240
NC, NS = 2, 16
NW = NC * NS
CH = 128
NCHUNK = E // CH
RND = NCHUNK // NW
XTRA = NCHUNK - RND * NW
RPT = NPAD // NS
APAD = 10112
ART = APAD // NS
RING = 3
NBE = 2
NBI = 4

_mesh = lambda: plsc.VectorSubcoreMesh(
    core_axis_name="c", subcore_axis_name="s", num_cores=NC, num_subcores=NS)


def _make_deg():
  @functools.partial(
      pl.kernel,
      out_type=jax.ShapeDtypeStruct((NC, NPAD), jnp.float32),
      mesh=_mesh(),
      scratch_types=[
          pltpu.VMEM((NBI, 2, CH), jnp.int32),
          pltpu.VMEM((NBI, CH), jnp.float32),
          pltpu.VMEM((RPT,), jnp.float32),
          pltpu.SemaphoreType.DMA((NBI,)),
          pltpu.SemaphoreType.DMA((NBI,)),
          pltpu.SemaphoreType.DMA((NBI,)),
          pltpu.VMEM_SHARED((NPAD,), jnp.float32),
      ],
  )
  def deg_kernel(ei, ew, deg_out, eib, ewb, zb, sei, sew, ssc, deg_sp):
    c = lax.axis_index("c")
    s = lax.axis_index("s")
    w = s * NC + c
    nch = RND + jnp.where(w < XTRA, 1, 0)

    @pl.loop(0, RPT // 16)
    def _zero(k):
      zb[pl.ds(k * 16, 16)] = jnp.zeros((16,), jnp.float32)

    pltpu.sync_copy(zb, deg_sp.at[pl.ds(s * RPT, RPT)])
    plsc.subcore_barrier()

    def fetch(r):
      b = lax.rem(r, NBI)
      ci = r * NW + w
      pltpu.async_copy(ei.at[:, pl.ds(ci * CH, CH)], eib.at[b], sei.at[b])
      pltpu.async_copy(ew.at[pl.ds(ci * CH, CH)], ewb.at[b], sew.at[b])

    def wait_fetch(b):
      pltpu.make_async_copy(ei.at[:, pl.ds(0, CH)], eib.at[b], sei.at[b]).wait()
      pltpu.make_async_copy(ew.at[pl.ds(0, CH)], ewb.at[b], sew.at[b]).wait()

    def wait_scat(b):
      pltpu.make_async_copy(ewb.at[b], deg_sp.at[eib.at[b, 1]],
                            ssc.at[b]).wait()

    fetch(0)

    @pl.loop(0, RND + 1)
    def _chunk(r):
      @pl.when(r < nch)
      def _():
        b = lax.rem(r, NBI)

        @pl.when(r >= NBI - 1)
        def _():
          wait_scat(lax.rem(r + 1, NBI))

        wait_fetch(b)

        @pl.when(r + 1 < nch)
        def _():
          fetch(r + 1)

        pltpu.async_copy(ewb.at[b], deg_sp.at[eib.at[b, 1]], ssc.at[b],
                         add=True)

    @pl.loop(0, NBI - 1)
    def _drain(k):
      r = nch - 1 - k

      @pl.when(r >= 0)
      def _():
        wait_scat(lax.rem(r, NBI))

    plsc.subcore_barrier()
    pltpu.sync_copy(deg_sp.at[pl.ds(s * RPT, RPT)],
                    deg_out.at[c, pl.ds(s * RPT, RPT)])

  return deg_kernel


def _make_agg():
  @functools.partial(
      pl.kernel,
      out_type=jax.ShapeDtypeStruct((NC, APAD, F), jnp.float32),
      mesh=_mesh(),
      compiler_params=pltpu.CompilerParams(needs_layout_passes=False),
      scratch_types=[
          pltpu.VMEM((RING, 2, CH), jnp.int32),
          pltpu.VMEM((NBE, CH), jnp.float32),
          pltpu.VMEM((RING, CH, F), jnp.float32),
          pltpu.SemaphoreType.DMA((RING,)),
          pltpu.SemaphoreType.DMA((NBE,)),
          pltpu.SemaphoreType.DMA((RING,)),
          pltpu.SemaphoreType.DMA((RING,)),
          pltpu.VMEM_SHARED((APAD, F), jnp.float32),
      ],
  )
  def agg_kernel(ei, ew, g, acc_out,
                 eib, ewb, rows, sei, sew, sg, ss, acc_sp):
    c = lax.axis_index("c")
    s = lax.axis_index("s")
    w = s * NC + c
    nch = RND + jnp.where(w < XTRA, 1, 0)

    @pl.loop(0, CH)
    def _zero(j):
      for k in range(F // 16):
        rows[0, j, pl.ds(k * 16, 16)] = jnp.zeros((16,), jnp.float32)

    for k in range(4):
      pltpu.sync_copy(rows.at[0], acc_sp.at[pl.ds(s * ART + k * CH, CH)])
    pltpu.sync_copy(rows.at[0, pl.ds(0, ART - 4 * CH)],
                    acc_sp.at[pl.ds(s * ART + 4 * CH, ART - 4 * CH)])
    plsc.subcore_barrier()

    def fetch(q):
      bi = lax.rem(q, RING)
      be = lax.rem(q, NBE)
      ci = q * NW + w
      pltpu.async_copy(ei.at[:, pl.ds(ci * CH, CH)], eib.at[bi], sei.at[bi])
      pltpu.async_copy(ew.at[pl.ds(ci * CH, CH)], ewb.at[be], sew.at[be])

    def wait_fetch(q):
      bi = lax.rem(q, RING)
      be = lax.rem(q, NBE)
      pltpu.make_async_copy(ei.at[:, pl.ds(0, CH)], eib.at[bi],
                            sei.at[bi]).wait()
      pltpu.make_async_copy(ew.at[pl.ds(0, CH)], ewb.at[be],
                            sew.at[be]).wait()

    def gather(q):
      b = lax.rem(q, RING)
      pltpu.async_copy(g.at[eib.at[b, 0]], rows.at[b], sg.at[b])

    def wait_gather(q):
      b = lax.rem(q, RING)
      pltpu.make_async_copy(g.at[eib.at[b, 0]], rows.at[b], sg.at[b]).wait()

    def wait_scat(q):
      b = lax.rem(q, RING)
      pltpu.make_async_copy(rows.at[b], acc_sp.at[eib.at[b, 1]],
                            ss.at[b]).wait()

    fetch(0)

    @pl.loop(0, RND + 1)
    def _chunk(r):
      @pl.when(r < nch)
      def _():
        b = lax.rem(r, RING)
        be = lax.rem(r, NBE)

        @pl.when(r == 0)
        def _():
          wait_fetch(0)
          gather(0)

        @pl.when(r >= RING - 1)
        def _():
          wait_scat(r + 1 - RING)

        @pl.when(r + 1 < nch)
        def _():
          fetch(r + 1)
          wait_fetch(r + 1)
          gather(r + 1)

        wait_gather(r)

        @pl.loop(0, CH, unroll=8)
        def _scale(j):
          wvec = plsc.load_gather(
              ewb, [jnp.full((16,), be, jnp.int32),
                    jnp.full((16,), j, jnp.int32)])
          for k in range(F // 16):
            rows[b, j, pl.ds(k * 16, 16)] = (
                rows[b, j, pl.ds(k * 16, 16)] * wvec)

        pltpu.async_copy(rows.at[b], acc_sp.at[eib.at[b, 1]], ss.at[b],
                         add=True)

    @pl.loop(0, RING - 1)
    def _drain(k):
      r = nch - 1 - k

      @pl.when(r >= 0)
      def _():
        wait_scat(r)

    plsc.subcore_barrier()
    pltpu.sync_copy(acc_sp.at[pl.ds(s * ART, ART)],
                    acc_out.at[c, pl.ds(s * ART, ART)])

  return agg_kernel


def _tc_pre(x, w, degp):
  B = 1000

  def body(x_ref, w_ref, degp_ref, g_ref):
    h = lax.dot_general(x_ref[...], w_ref[...],
                        (((1,), (1,)), ((), ())),
                        preferred_element_type=jnp.float32)
    deg = degp_ref[0] + degp_ref[1] + 1.0
    dis = lax.rsqrt(deg)
    g_ref[...] = h * dis

  return pl.pallas_call(
      body,
      grid=(N // B,),
      in_specs=[
          pl.BlockSpec((B, F), lambda i: (i, 0)),
          pl.BlockSpec((F, F), lambda i: (0, 0)),
          pl.BlockSpec((NC, B, 1), lambda i: (0, i, 0)),
      ],
      out_specs=pl.BlockSpec((B, F), lambda i: (i, 0)),
      out_shape=jax.ShapeDtypeStruct((N, F), jnp.float32),
  )(x, w, degp)


def _tc_fin(accp, g, degp, b2):
  B = 400

  def body(accp_ref, g_ref, degp_ref, b_ref, x4_ref, emb_ref):
    acc = accp_ref[0] + accp_ref[1]
    deg = degp_ref[0] + degp_ref[1] + 1.0
    dis = lax.rsqrt(deg)
    out = (acc + g_ref[...]) * dis + b_ref[...]
    out = jnp.maximum(out, 0.0)
    emb_ref[...] = out
    x4_ref[...] = lax.broadcast_in_dim(out, (B, HREP, NUM_HEADS, F), (0, 3))

  return pl.pallas_call(
      body,
      grid=(N // B,),
      in_specs=[
          pl.BlockSpec((NC, B, F), lambda i: (0, i, 0)),
          pl.BlockSpec((B, F), lambda i: (i, 0)),
          pl.BlockSpec((NC, B, 1), lambda i: (0, i, 0)),
          pl.BlockSpec((1, F), lambda i: (0, 0)),
      ],
      out_specs=[
          pl.BlockSpec((B, HREP, NUM_HEADS, F), lambda i: (i, 0, 0, 0)),
          pl.BlockSpec((B, F), lambda i: (i, 0)),
      ],
      out_shape=[
          jax.ShapeDtypeStruct((N, HREP, NUM_HEADS, F), jnp.float32),
          jax.ShapeDtypeStruct((N, F), jnp.float32),
      ],
  )(accp, g, degp, b2)


_deg_kernel = _make_deg()
_agg_kernel = _make_agg()


def kernel(x, edge_index, edge_weight, coordinate, W, b):
  del coordinate
  ei = edge_index.astype(jnp.int32)
  degp = _deg_kernel(ei, edge_weight).reshape(NC, NPAD, 1)
  g = _tc_pre(x, W, degp)
  accp = _agg_kernel(ei, edge_weight, g)
  x4_p, emb = _tc_fin(accp, g, degp, b.reshape(1, F))
  return (x4_p.transpose(0, 2, 3, 1), emb)


# speedup vs baseline: 2.0552x; 2.0552x over previous
import functools

import jax
import jax.numpy as jnp
from jax import lax
from jax.experimental import pallas as pl
from jax.experimental.pallas import tpu as pltpu
from jax.experimental.pallas import tpu_sc as plsc

N = 10000
F = 128
E = 320000
NUM_HEADS = 4
HREP = 12
NPAD = 10240
NC, NS = 2, 16
NW = NC * NS
CH = 128
NCHUNK = E // CH
RND = NCHUNK // NW
XTRA = NCHUNK - RND * NW
RPT = NPAD // NS
NBI = 4
NBR = 2

_mesh = lambda: plsc.VectorSubcoreMesh(
    core_axis_name="c", subcore_axis_name="s", num_cores=NC, num_subcores=NS)


def _make_deg():
  SPAN = RND * CH
  NT = NPAD // NS

  @functools.partial(
      pl.kernel,
      out_type=jax.ShapeDtypeStruct((NC, NPAD), jnp.float32),
      mesh=_mesh(),
      compiler_params=pltpu.CompilerParams(needs_layout_passes=False),
      scratch_types=[
          pltpu.VMEM((2, SPAN), jnp.int32),
          pltpu.VMEM((SPAN,), jnp.float32),
          pltpu.VMEM((2, CH), jnp.int32),
          pltpu.VMEM((CH,), jnp.float32),
          pltpu.VMEM((NPAD,), jnp.float32),
          pltpu.VMEM((NS, NPAD // NS), jnp.float32),
          pltpu.VMEM((NPAD // NS,), jnp.float32),
          pltpu.SemaphoreType.DMA,
          pltpu.VMEM_SHARED((NS, NPAD), jnp.float32),
      ],
  )
  def deg_kernel(ei, ew, deg_out, ebuf, ewbuf, ebx, ewbx, priv, lbuf, res,
                 sem, stage):
    c = lax.axis_index("c")
    s = lax.axis_index("s")
    w = s * NC + c
    start = (RND * w + jnp.minimum(w, XTRA)) * CH

    pltpu.async_copy(ei.at[:, pl.ds(start, SPAN)], ebuf, sem)
    pltpu.async_copy(ew.at[pl.ds(start, SPAN)], ewbuf, sem)

    @pl.when(w < XTRA)
    def _():
      pltpu.async_copy(ei.at[:, pl.ds(start + SPAN, CH)], ebx, sem)
      pltpu.async_copy(ew.at[pl.ds(start + SPAN, CH)], ewbx, sem)

    @pl.loop(0, NPAD // 16, unroll=8)
    def _zero(i):
      priv[pl.ds(i * 16, 16)] = jnp.zeros((16,), jnp.float32)

    pltpu.make_async_copy(ei.at[:, pl.ds(0, SPAN)], ebuf, sem).wait()
    pltpu.make_async_copy(ew.at[pl.ds(0, SPAN)], ewbuf, sem).wait()

    @pl.when(w < XTRA)
    def _():
      pltpu.make_async_copy(ei.at[:, pl.ds(0, CH)], ebx, sem).wait()
      pltpu.make_async_copy(ew.at[pl.ds(0, CH)], ewbx, sem).wait()

    @pl.loop(0, SPAN // 16, unroll=8)
    def _acc(i):
      idxv = ebuf[1, pl.ds(i * 16, 16)]
      ewv = ewbuf[pl.ds(i * 16, 16)]
      plsc.addupdate_scatter(priv, [idxv], ewv)

    @pl.when(w < XTRA)
    def _():
      @pl.loop(0, CH // 16)
      def _accx(i):
        idxv = ebx[1, pl.ds(i * 16, 16)]
        ewv = ewbx[pl.ds(i * 16, 16)]
        plsc.addupdate_scatter(priv, [idxv], ewv)

    pltpu.sync_copy(priv, stage.at[s])
    plsc.subcore_barrier()
    pltpu.sync_copy(stage.at[:, pl.ds(s * NT, NT)], lbuf)

    @pl.loop(0, NT // 16, unroll=4)
    def _red(i):
      acc = lbuf[0, pl.ds(i * 16, 16)]
      for t in range(1, NS):
        acc = acc + lbuf[t, pl.ds(i * 16, 16)]
      res[pl.ds(i * 16, 16)] = acc

    pltpu.sync_copy(res, deg_out.at[c, pl.ds(s * NT, NT)])

  return deg_kernel


def _make_agg():
  @functools.partial(
      pl.kernel,
      out_type=jax.ShapeDtypeStruct((NC, NPAD, F), jnp.float32),
      mesh=_mesh(),
      compiler_params=pltpu.CompilerParams(needs_layout_passes=False),
      scratch_types=[
          pltpu.VMEM((NBI, 2, CH), jnp.int32),
          pltpu.VMEM((NBI, CH), jnp.float32),
          pltpu.VMEM((NBR, CH, F), jnp.float32),
          pltpu.SemaphoreType.DMA((NBI,)),
          pltpu.SemaphoreType.DMA((NBI,)),
          pltpu.SemaphoreType.DMA((NBR,)),
          pltpu.SemaphoreType.DMA((NBR,)),
          pltpu.VMEM_SHARED((NPAD, F), jnp.float32),
      ],
  )
  def agg_kernel(ei, ew, g, acc_out,
                 eib, ewb, rows, sei, sew, sg, ss, acc_sp):
    c = lax.axis_index("c")
    s = lax.axis_index("s")
    w = s * NC + c
    nch = RND + jnp.where(w < XTRA, 1, 0)

    @pl.loop(0, CH)
    def _zero(j):
      for k in range(F // 16):
        rows[0, j, pl.ds(k * 16, 16)] = jnp.zeros((16,), jnp.float32)

    for k in range(RPT // CH):
      pltpu.sync_copy(rows.at[0], acc_sp.at[pl.ds(s * RPT + k * CH, CH)])
    plsc.subcore_barrier()

    def fetch(r):
      b = lax.rem(r, NBI)
      ci = r * NW + w
      pltpu.async_copy(ei.at[:, pl.ds(ci * CH, CH)], eib.at[b], sei.at[b])
      pltpu.async_copy(ew.at[pl.ds(ci * CH, CH)], ewb.at[b], sew.at[b])

    def wait_fetch(b):
      pltpu.make_async_copy(ei.at[:, pl.ds(0, CH)], eib.at[b], sei.at[b]).wait()
      pltpu.make_async_copy(ew.at[pl.ds(0, CH)], ewb.at[b], sew.at[b]).wait()

    def gather(r):
      bi = lax.rem(r, NBI)
      br = lax.rem(r, NBR)
      pltpu.async_copy(g.at[eib.at[bi, 0]], rows.at[br], sg.at[br])

    def wait_gather(r):
      bi = lax.rem(r, NBI)
      br = lax.rem(r, NBR)
      pltpu.make_async_copy(g.at[eib.at[bi, 0]], rows.at[br], sg.at[br]).wait()

    def wait_scat(q):
      bi = lax.rem(q, NBI)
      br = lax.rem(q, NBR)
      pltpu.make_async_copy(rows.at[br], acc_sp.at[eib.at[bi, 1]],
                            ss.at[br]).wait()

    fetch(0)

    @pl.when(nch > 1)
    def _():
      fetch(1)

    @pl.loop(0, RND + 1)
    def _chunk(r):
      @pl.when(r < nch)
      def _():
        bi = lax.rem(r, NBI)
        br = lax.rem(r, NBR)

        @pl.when(r == 0)
        def _():
          wait_fetch(bi)
          gather(0)

        @pl.when(r >= NBR - 1)
        def _():
          wait_scat(r + 1 - NBR)

        @pl.when(r + 1 < nch)
        def _():
          wait_fetch(lax.rem(r + 1, NBI))
          gather(r + 1)

        @pl.when(r + 2 < nch)
        def _():
          fetch(r + 2)

        wait_gather(r)

        @pl.loop(0, CH, unroll=8)
        def _scale(j):
          wvec = plsc.load_gather(
              ewb, [jnp.full((16,), bi, jnp.int32),
                    jnp.full((16,), j, jnp.int32)])
          for k in range(F // 16):
            rows[br, j, pl.ds(k * 16, 16)] = (
                rows[br, j, pl.ds(k * 16, 16)] * wvec)

        pltpu.async_copy(rows.at[br], acc_sp.at[eib.at[bi, 1]], ss.at[br],
                         add=True)

    @pl.loop(0, NBR - 1)
    def _drain(k):
      r = nch - 1 - k

      @pl.when(r >= 0)
      def _():
        wait_scat(r)

    plsc.subcore_barrier()
    pltpu.sync_copy(acc_sp.at[pl.ds(s * RPT, RPT)],
                    acc_out.at[c, pl.ds(s * RPT, RPT)])

  return agg_kernel


def _tc_pre(x, w, degp):
  B = 1000

  def body(x_ref, w_ref, degp_ref, g_ref):
    h = lax.dot_general(x_ref[...], w_ref[...],
                        (((1,), (1,)), ((), ())),
                        preferred_element_type=jnp.float32)
    deg = degp_ref[0] + degp_ref[1] + 1.0
    dis = lax.rsqrt(deg)
    g_ref[...] = h * dis

  return pl.pallas_call(
      body,
      grid=(N // B,),
      in_specs=[
          pl.BlockSpec((B, F), lambda i: (i, 0)),
          pl.BlockSpec((F, F), lambda i: (0, 0)),
          pl.BlockSpec((NC, B, 1), lambda i: (0, i, 0)),
      ],
      out_specs=pl.BlockSpec((B, F), lambda i: (i, 0)),
      out_shape=jax.ShapeDtypeStruct((N, F), jnp.float32),
  )(x, w, degp)


def _tc_fin(accp, g, degp, b2):
  B = 400

  def body(accp_ref, g_ref, degp_ref, b_ref, x4_ref, emb_ref):
    acc = accp_ref[0] + accp_ref[1]
    deg = degp_ref[0] + degp_ref[1] + 1.0
    dis = lax.rsqrt(deg)
    out = (acc + g_ref[...]) * dis + b_ref[...]
    out = jnp.maximum(out, 0.0)
    emb_ref[...] = out
    x4_ref[...] = lax.broadcast_in_dim(out, (B, HREP, NUM_HEADS, F), (0, 3))

  return pl.pallas_call(
      body,
      grid=(N // B,),
      in_specs=[
          pl.BlockSpec((NC, B, F), lambda i: (0, i, 0)),
          pl.BlockSpec((B, F), lambda i: (i, 0)),
          pl.BlockSpec((NC, B, 1), lambda i: (0, i, 0)),
          pl.BlockSpec((1, F), lambda i: (0, 0)),
      ],
      out_specs=[
          pl.BlockSpec((B, HREP, NUM_HEADS, F), lambda i: (i, 0, 0, 0)),
          pl.BlockSpec((B, F), lambda i: (i, 0)),
      ],
      out_shape=[
          jax.ShapeDtypeStruct((N, HREP, NUM_HEADS, F), jnp.float32),
          jax.ShapeDtypeStruct((N, F), jnp.float32),
      ],
  )(accp, g, degp, b2)


_deg_kernel = _make_deg()
_agg_kernel = _make_agg()


def kernel(x, edge_index, edge_weight, coordinate, W, b):
  del coordinate
  ei = edge_index.astype(jnp.int32)
  degp = _deg_kernel(ei, edge_weight).reshape(NC, NPAD, 1)
  g = _tc_pre(x, W, degp)
  accp = _agg_kernel(ei, edge_weight, g)
  x4_p, emb = _tc_fin(accp, g, degp, b.reshape(1, F))
  return (x4_p.transpose(0, 2, 3, 1), emb)


# speedup vs baseline: 2.0822x; 1.0131x over previous
import functools

import jax
import jax.numpy as jnp
from jax import lax
from jax.experimental import pallas as pl
from jax.experimental.pallas import tpu as pltpu
from jax.experimental.pallas import tpu_sc as plsc

N = 10000
F = 128
E = 320000
NUM_HEADS = 4
HREP = 12
NPAD = 10240
NC, NS = 2, 16
NW = NC * NS
CH = 128
NCHUNK = E // CH
RND = NCHUNK // NW
XTRA = NCHUNK - RND * NW
RPT = NPAD // NS
NBI = 4
NBR = 2

_mesh = lambda: plsc.VectorSubcoreMesh(
    core_axis_name="c", subcore_axis_name="s", num_cores=NC, num_subcores=NS)


def _make_deg():
  SPAN = RND * CH
  NT = NPAD // NS

  @functools.partial(
      pl.kernel,
      out_type=jax.ShapeDtypeStruct((NC, NPAD), jnp.float32),
      mesh=_mesh(),
      compiler_params=pltpu.CompilerParams(needs_layout_passes=False),
      scratch_types=[
          pltpu.VMEM((2, SPAN), jnp.int32),
          pltpu.VMEM((SPAN,), jnp.float32),
          pltpu.VMEM((2, CH), jnp.int32),
          pltpu.VMEM((CH,), jnp.float32),
          pltpu.VMEM((NPAD,), jnp.float32),
          pltpu.VMEM((NS, NPAD // NS), jnp.float32),
          pltpu.VMEM((NPAD // NS,), jnp.float32),
          pltpu.SemaphoreType.DMA,
          pltpu.VMEM_SHARED((NS, NPAD), jnp.float32),
      ],
  )
  def deg_kernel(ei, ew, deg_out, ebuf, ewbuf, ebx, ewbx, priv, lbuf, res,
                 sem, stage):
    c = lax.axis_index("c")
    s = lax.axis_index("s")
    w = s * NC + c
    start = (RND * w + jnp.minimum(w, XTRA)) * CH

    pltpu.async_copy(ei.at[:, pl.ds(start, SPAN)], ebuf, sem)
    pltpu.async_copy(ew.at[pl.ds(start, SPAN)], ewbuf, sem)

    @pl.when(w < XTRA)
    def _():
      pltpu.async_copy(ei.at[:, pl.ds(start + SPAN, CH)], ebx, sem)
      pltpu.async_copy(ew.at[pl.ds(start + SPAN, CH)], ewbx, sem)

    @pl.loop(0, NPAD // 16, unroll=8)
    def _zero(i):
      priv[pl.ds(i * 16, 16)] = jnp.zeros((16,), jnp.float32)

    pltpu.make_async_copy(ei.at[:, pl.ds(0, SPAN)], ebuf, sem).wait()
    pltpu.make_async_copy(ew.at[pl.ds(0, SPAN)], ewbuf, sem).wait()

    @pl.when(w < XTRA)
    def _():
      pltpu.make_async_copy(ei.at[:, pl.ds(0, CH)], ebx, sem).wait()
      pltpu.make_async_copy(ew.at[pl.ds(0, CH)], ewbx, sem).wait()

    @pl.loop(0, SPAN // 16, unroll=8)
    def _acc(i):
      idxv = ebuf[1, pl.ds(i * 16, 16)]
      ewv = ewbuf[pl.ds(i * 16, 16)]
      plsc.addupdate_scatter(priv, [idxv], ewv)

    @pl.when(w < XTRA)
    def _():
      @pl.loop(0, CH // 16)
      def _accx(i):
        idxv = ebx[1, pl.ds(i * 16, 16)]
        ewv = ewbx[pl.ds(i * 16, 16)]
        plsc.addupdate_scatter(priv, [idxv], ewv)

    pltpu.sync_copy(priv, stage.at[s])
    plsc.subcore_barrier()
    pltpu.sync_copy(stage.at[:, pl.ds(s * NT, NT)], lbuf)

    @pl.loop(0, NT // 16, unroll=4)
    def _red(i):
      acc = lbuf[0, pl.ds(i * 16, 16)]
      for t in range(1, NS):
        acc = acc + lbuf[t, pl.ds(i * 16, 16)]
      res[pl.ds(i * 16, 16)] = acc

    pltpu.sync_copy(res, deg_out.at[c, pl.ds(s * NT, NT)])

  return deg_kernel


def _make_agg():
  @functools.partial(
      pl.kernel,
      out_type=jax.ShapeDtypeStruct((NC, NPAD, F), jnp.float32),
      mesh=_mesh(),
      compiler_params=pltpu.CompilerParams(needs_layout_passes=False),
      scratch_types=[
          pltpu.VMEM((NBI, 2, CH), jnp.int32),
          pltpu.VMEM((NBI, CH), jnp.float32),
          pltpu.VMEM((NBR, CH, F), jnp.float32),
          pltpu.VMEM((NPAD,), jnp.float32),
          pltpu.VMEM((CH,), jnp.float32),
          pltpu.SemaphoreType.DMA((NBI,)),
          pltpu.SemaphoreType.DMA((NBI,)),
          pltpu.SemaphoreType.DMA((NBR,)),
          pltpu.SemaphoreType.DMA((NBR,)),
          pltpu.VMEM_SHARED((NPAD, F), jnp.float32),
      ],
  )
  def agg_kernel(ei, ew, g, dis2, acc_out,
                 eib, ewb, rows, disl, cw, sei, sew, sg, ss, acc_sp):
    c = lax.axis_index("c")
    s = lax.axis_index("s")
    w = s * NC + c
    nch = RND + jnp.where(w < XTRA, 1, 0)

    pltpu.sync_copy(dis2.at[0], disl)

    @pl.loop(0, CH)
    def _zero(j):
      for k in range(F // 16):
        rows[0, j, pl.ds(k * 16, 16)] = jnp.zeros((16,), jnp.float32)

    for k in range(RPT // CH):
      pltpu.sync_copy(rows.at[0], acc_sp.at[pl.ds(s * RPT + k * CH, CH)])
    plsc.subcore_barrier()

    def fetch(r):
      b = lax.rem(r, NBI)
      ci = r * NW + w
      pltpu.async_copy(ei.at[:, pl.ds(ci * CH, CH)], eib.at[b], sei.at[b])
      pltpu.async_copy(ew.at[pl.ds(ci * CH, CH)], ewb.at[b], sew.at[b])

    def wait_fetch(b):
      pltpu.make_async_copy(ei.at[:, pl.ds(0, CH)], eib.at[b], sei.at[b]).wait()
      pltpu.make_async_copy(ew.at[pl.ds(0, CH)], ewb.at[b], sew.at[b]).wait()

    def gather(r):
      bi = lax.rem(r, NBI)
      br = lax.rem(r, NBR)
      pltpu.async_copy(g.at[eib.at[bi, 0]], rows.at[br], sg.at[br])

    def wait_gather(r):
      bi = lax.rem(r, NBI)
      br = lax.rem(r, NBR)
      pltpu.make_async_copy(g.at[eib.at[bi, 0]], rows.at[br], sg.at[br]).wait()

    def wait_scat(q):
      bi = lax.rem(q, NBI)
      br = lax.rem(q, NBR)
      pltpu.make_async_copy(rows.at[br], acc_sp.at[eib.at[bi, 1]],
                            ss.at[br]).wait()

    fetch(0)

    @pl.when(nch > 1)
    def _():
      fetch(1)

    @pl.loop(0, RND + 1)
    def _chunk(r):
      @pl.when(r < nch)
      def _():
        bi = lax.rem(r, NBI)
        br = lax.rem(r, NBR)

        @pl.when(r == 0)
        def _():
          wait_fetch(bi)
          gather(0)

        @pl.when(r >= NBR - 1)
        def _():
          wait_scat(r + 1 - NBR)

        @pl.when(r + 1 < nch)
        def _():
          wait_fetch(lax.rem(r + 1, NBI))
          gather(r + 1)

        @pl.when(r + 2 < nch)
        def _():
          fetch(r + 2)

        wait_gather(r)

        @pl.loop(0, CH // 16, unroll=4)
        def _cw(g16):
          rv = eib[bi, 0, pl.ds(g16 * 16, 16)]
          dvec = plsc.load_gather(disl, [rv])
          cw[pl.ds(g16 * 16, 16)] = dvec * ewb[bi, pl.ds(g16 * 16, 16)]

        @pl.loop(0, CH, unroll=8)
        def _scale(j):
          wvec = plsc.load_gather(cw, [jnp.full((16,), j, jnp.int32)])
          for k in range(F // 16):
            rows[br, j, pl.ds(k * 16, 16)] = (
                rows[br, j, pl.ds(k * 16, 16)] * wvec)

        pltpu.async_copy(rows.at[br], acc_sp.at[eib.at[bi, 1]], ss.at[br],
                         add=True)

    @pl.loop(0, NBR - 1)
    def _drain(k):
      r = nch - 1 - k

      @pl.when(r >= 0)
      def _():
        wait_scat(r)

    plsc.subcore_barrier()
    pltpu.sync_copy(acc_sp.at[pl.ds(s * RPT, RPT)],
                    acc_out.at[c, pl.ds(s * RPT, RPT)])

  return agg_kernel


def _tc_h(x, w):
  B = 1000

  def body(x_ref, w_ref, h_ref):
    h_ref[...] = lax.dot_general(x_ref[...], w_ref[...],
                                 (((1,), (1,)), ((), ())),
                                 preferred_element_type=jnp.float32)

  return pl.pallas_call(
      body,
      grid=(N // B,),
      in_specs=[
          pl.BlockSpec((B, F), lambda i: (i, 0)),
          pl.BlockSpec((F, F), lambda i: (0, 0)),
      ],
      out_specs=pl.BlockSpec((B, F), lambda i: (i, 0)),
      out_shape=jax.ShapeDtypeStruct((N, F), jnp.float32),
  )(x, w)


def _tc_dis(degp):
  def body(degp_ref, dis_ref):
    deg = degp_ref[0] + degp_ref[1] + 1.0
    dis_ref[...] = lax.rsqrt(deg)[None, :]

  return pl.pallas_call(
      body,
      in_specs=[pl.BlockSpec((NC, NPAD), lambda: (0, 0))],
      out_specs=pl.BlockSpec((1, NPAD), lambda: (0, 0)),
      out_shape=jax.ShapeDtypeStruct((1, NPAD), jnp.float32),
  )(degp)


def _tc_fin(accp, g, degp, b2):
  B = 400

  def body(accp_ref, h_ref, degp_ref, b_ref, x4_ref, emb_ref):
    acc = accp_ref[0] + accp_ref[1]
    deg = degp_ref[0] + degp_ref[1] + 1.0
    dis = lax.rsqrt(deg)
    out = (acc + h_ref[...] * dis) * dis + b_ref[...]
    out = jnp.maximum(out, 0.0)
    emb_ref[...] = out
    x4_ref[...] = lax.broadcast_in_dim(out, (B, HREP, NUM_HEADS, F), (0, 3))

  return pl.pallas_call(
      body,
      grid=(N // B,),
      in_specs=[
          pl.BlockSpec((NC, B, F), lambda i: (0, i, 0)),
          pl.BlockSpec((B, F), lambda i: (i, 0)),
          pl.BlockSpec((NC, B, 1), lambda i: (0, i, 0)),
          pl.BlockSpec((1, F), lambda i: (0, 0)),
      ],
      out_specs=[
          pl.BlockSpec((B, HREP, NUM_HEADS, F), lambda i: (i, 0, 0, 0)),
          pl.BlockSpec((B, F), lambda i: (i, 0)),
      ],
      out_shape=[
          jax.ShapeDtypeStruct((N, HREP, NUM_HEADS, F), jnp.float32),
          jax.ShapeDtypeStruct((N, F), jnp.float32),
      ],
  )(accp, g, degp, b2)


_deg_kernel = _make_deg()
_agg_kernel = _make_agg()


def kernel(x, edge_index, edge_weight, coordinate, W, b):
  del coordinate
  ei = edge_index.astype(jnp.int32)
  degp = _deg_kernel(ei, edge_weight)
  h = _tc_h(x, W)
  dis2 = _tc_dis(degp)
  accp = _agg_kernel(ei, edge_weight, h, dis2)
  x4_p, emb = _tc_fin(accp, h, degp.reshape(NC, NPAD, 1), b.reshape(1, F))
  return (x4_p.transpose(0, 2, 3, 1), emb)
